# Initial kernel scaffold; baseline (speedup 1.0000x reference)
import jax
import jax.numpy as jnp
from jax.experimental import pallas as pl


def kernel(node_feature, edge_index, W1, b1, W2, b2):
    raise NotImplementedError("write your pallas kernel here")


# speedup vs baseline: 5.3372x; 5.3372x over previous
import functools

import jax
import jax.numpy as jnp
from jax import lax
from jax.experimental import pallas as pl
from jax.experimental.pallas import tpu as pltpu
from jax.experimental.pallas import tpu_sc as plsc

N = 10000
D = 128
H = 128
DEG = 32

NC = 2
NS = 16
NW = NC * NS

CHUNK_D = 4
CHUNK_E = CHUNK_D * DEG
NUM_CHUNKS = N // CHUNK_D


def _mm_body(x_ref, w1a_ref, w1b_ref, b1_ref, a_ref, b_ref):
    x = x_ref[...]
    a_ref[...] = (
        jnp.dot(x, w1a_ref[...], preferred_element_type=jnp.float32)
        + b1_ref[...]
    )
    b_ref[...] = jnp.dot(x, w1b_ref[...], preferred_element_type=jnp.float32)


def _node_transform(x, w1a, w1b, b1):
    blk = 2000
    return pl.pallas_call(
        _mm_body,
        grid=(N // blk,),
        in_specs=[
            pl.BlockSpec((blk, D), lambda i: (i, 0)),
            pl.BlockSpec((D, H), lambda i: (0, 0)),
            pl.BlockSpec((D, H), lambda i: (0, 0)),
            pl.BlockSpec((1, H), lambda i: (0, 0)),
        ],
        out_specs=[
            pl.BlockSpec((blk, H), lambda i: (i, 0)),
            pl.BlockSpec((blk, H), lambda i: (i, 0)),
        ],
        out_shape=[
            jax.ShapeDtypeStruct((N, H), jnp.float32),
            jax.ShapeDtypeStruct((N, H), jnp.float32),
        ],
    )(x, w1a, w1b, b1)


_DNUMS = lax.GatherDimensionNumbers(
    offset_dims=(), collapsed_slice_dims=(0,), start_index_map=(0,)
)


def _perm(v, idx):
    return lax.gather(
        v, idx[:, None], _DNUMS, (1,),
        mode=lax.GatherScatterMode.PROMISE_IN_BOUNDS,
    )


def _edge_body(a_hbm, b_hbm, src_hbm, w2_hbm, b2_hbm, out_hbm,
               idx_v, rows_v, a_v, out_v, w2_v, b2_v, sem):
    cid = lax.axis_index("c")
    sid = lax.axis_index("s")
    wid = sid * NC + cid

    pltpu.sync_copy(w2_hbm, w2_v)
    pltpu.sync_copy(b2_hbm, b2_v)
    b2vec = b2_v[...]
    iota16 = lax.iota(jnp.int32, 16)
    w_chunks = [w2_v[pl.ds(16 * jb, 16)] for jb in range(H // 16)]
    strides = (1, 2, 4, 8)
    perm_idx = [iota16 ^ s for s in strides]
    masks = [(iota16 & s) == 0 for s in strides]

    trips = (NUM_CHUNKS - wid + NW - 1) // NW

    def chunk_body(i, _):
        c = wid + i * NW
        pltpu.sync_copy(src_hbm.at[pl.ds(c * CHUNK_E, CHUNK_E)], idx_v)
        pltpu.sync_copy(a_hbm.at[pl.ds(c * CHUNK_D, CHUNK_D)], a_v)
        pltpu.async_copy(b_hbm.at[idx_v], rows_v, sem).wait()

        def dst_body(d, _):
            a_chunks = [a_v[d, pl.ds(16 * jb, 16)] for jb in range(H // 16)]
            for half in range(DEG // 16):
                level = []
                for e in range(16):
                    row = d * DEG + half * 16 + e
                    acc = None
                    for jb in range(H // 16):
                        g = rows_v[row, pl.ds(16 * jb, 16)]
                        t = g + a_chunks[jb]
                        u = jnp.maximum(t, t * 0.01)
                        m = u * w_chunks[jb]
                        acc = m if acc is None else acc + m
                    level.append(acc)
                for li in range(4):
                    nxt = []
                    for p in range(0, len(level), 2):
                        va, vb = level[p], level[p + 1]
                        hi = jnp.where(masks[li], va, vb)
                        lo = jnp.where(masks[li], vb, va)
                        nxt.append(hi + _perm(lo, perm_idx[li]))
                    level = nxt
                out_v[d, pl.ds(half * 16, 16)] = level[0] + b2vec
            return 0

        lax.fori_loop(0, CHUNK_D, dst_body, 0)
        pltpu.sync_copy(out_v, out_hbm.at[pl.ds(c * CHUNK_D, CHUNK_D)])
        return 0

    lax.fori_loop(0, trips, chunk_body, 0)


_edge_call = functools.partial(
    pl.kernel,
    mesh=plsc.VectorSubcoreMesh(core_axis_name="c", subcore_axis_name="s"),
    out_type=jax.ShapeDtypeStruct((N, DEG), jnp.float32),
    scratch_types=[
        pltpu.VMEM((CHUNK_E,), jnp.int32),
        pltpu.VMEM((CHUNK_E, H), jnp.float32),
        pltpu.VMEM((CHUNK_D, H), jnp.float32),
        pltpu.VMEM((CHUNK_D, DEG), jnp.float32),
        pltpu.VMEM((H,), jnp.float32),
        pltpu.VMEM((16,), jnp.float32),
        pltpu.SemaphoreType.DMA,
    ],
)(_edge_body)


def kernel(node_feature, edge_index, W1, b1, W2, b2):
    src = edge_index[0]
    a, b = _node_transform(
        node_feature, W1[:D], W1[D:], b1.reshape(1, H)
    )
    w2 = W2.reshape(H)
    b2v = jnp.broadcast_to(b2.reshape(1), (16,)).astype(jnp.float32)
    return _edge_call(a, b, src, w2, b2v)
